# baseline (speedup 1.0000x reference)
import functools

import jax
import jax.numpy as jnp
from jax import lax
from jax.experimental import pallas as pl
from jax.experimental.pallas import tpu as pltpu
from jax.experimental.pallas import tpu_sc as plsc

N_ROWS = 100000
DIM = 128
K = 4
EPS = 1e-6

NC = 2
NS = 16
L = 16
NW = NC * NS

N_SC = 68608
ROWS_PER_W = N_SC // NW
CHUNK = 256
GROUPS = CHUNK // L
FULL_CHUNKS = ROWS_PER_W // CHUNK
TAIL = ROWS_PER_W - FULL_CHUNKS * CHUNK
TAIL_GROUPS = -(-TAIL // L)
PER_W = 80
F32_INF = float("inf")

BLK = 1024
TC_BLK_OFF = N_SC // BLK
TCN = N_ROWS - N_SC
TGRID = -(-TCN // BLK)


def _sqrt16(x):
    x = jnp.maximum(x, jnp.float32(1e-30))
    i = lax.bitcast_convert_type(x, jnp.int32)
    i = jnp.int32(0x5F3759DF) - lax.shift_right_logical(i, 1)
    y = lax.bitcast_convert_type(i, jnp.float32)
    half = jnp.float32(0.5)
    three_half = jnp.float32(1.5)
    for _ in range(3):
        y = y * (three_half - half * x * y * y)
    return x * y


def _insert4(t0, t1, t2, t3, c):
    u0 = jnp.minimum(t0, c)
    c = jnp.maximum(t0, c)
    u1 = jnp.minimum(t1, c)
    c = jnp.maximum(t1, c)
    u2 = jnp.minimum(t2, c)
    c = jnp.maximum(t2, c)
    u3 = jnp.minimum(t3, c)
    return u0, u1, u2, u3


def _chunk_d2(buf, wbuf, ngroups, row_base_vec=None):
    lane = lax.iota(jnp.int32, L)
    if row_base_vec is None:
        row_base_vec = [lane * jnp.int32(DIM)] * ngroups
    shared = all(rb is row_base_vec[0] for rb in row_base_vec)
    grefs = [buf.at[pl.ds(g * L * DIM, L * DIM)] for g in range(ngroups)]

    def body(j, carry):
        cvec = carry[0]
        accs = list(carry[1:])
        wv = plsc.load_gather(wbuf, [cvec])
        idx0 = row_base_vec[0] + cvec
        for g in range(ngroups):
            idx = idx0 if shared else row_base_vec[g] + cvec
            col = plsc.load_gather(grefs[g], [idx])
            t = col - wv
            accs[g] = accs[g] + t * t
        cvec = jnp.bitwise_and(cvec + 1, jnp.int32(DIM - 1))
        return (cvec,) + tuple(accs)

    init = (lane,) + tuple(jnp.zeros((L,), jnp.float32) for _ in range(ngroups))
    res = lax.fori_loop(0, DIM, body, init, unroll=2)
    return list(res[1:])


def _sc_body(vectors_hbm, g_hbm, out_hbm, wbuf, buf0, buf1, tailbuf,
             staging, sem0, sem1, sem2):
    wid = lax.axis_index("s") * NC + lax.axis_index("c")
    base = wid * (ROWS_PER_W * DIM)
    lane = lax.iota(jnp.int32, L)
    chunk_w = CHUNK * DIM
    tail_w = TAIL * DIM

    pltpu.sync_copy(g_hbm, wbuf)
    for k in range(DIM // L):
        sl = pl.ds(k * L, L)
        wbuf[sl] = wbuf[sl] - jnp.float32(EPS)
    pltpu.async_copy(vectors_hbm.at[pl.ds(base, chunk_w)], buf0, sem0)
    pltpu.async_copy(vectors_hbm.at[pl.ds(base + chunk_w, chunk_w)], buf1, sem1)
    pltpu.async_copy(
        vectors_hbm.at[pl.ds(base + FULL_CHUNKS * chunk_w, tail_w)],
        tailbuf.at[pl.ds(0, tail_w)], sem2)

    def process_full(buf, state):
        s_sum, t0, t1, t2, t3 = state
        accs = _chunk_d2(buf, wbuf, GROUPS)
        for g in range(GROUPS):
            d2 = accs[g]
            t0, t1, t2, t3 = _insert4(t0, t1, t2, t3, d2)
            s_sum = s_sum + _sqrt16(d2)
        return (s_sum, t0, t1, t2, t3)

    def loop_body(i, state):
        c0 = 2 * i
        pltpu.make_async_copy(
            vectors_hbm.at[pl.ds(base + c0 * chunk_w, chunk_w)],
            buf0, sem0).wait()
        state = process_full(buf0, state)

        @pl.when(c0 + 2 < FULL_CHUNKS)
        def _():
            pltpu.async_copy(
                vectors_hbm.at[pl.ds(base + (c0 + 2) * chunk_w, chunk_w)],
                buf0, sem0)

        pltpu.make_async_copy(
            vectors_hbm.at[pl.ds(base + (c0 + 1) * chunk_w, chunk_w)],
            buf1, sem1).wait()
        state = process_full(buf1, state)

        @pl.when(c0 + 3 < FULL_CHUNKS)
        def _():
            pltpu.async_copy(
                vectors_hbm.at[pl.ds(base + (c0 + 3) * chunk_w, chunk_w)],
                buf1, sem1)

        return state

    zero = jnp.zeros((L,), jnp.float32)
    inf = jnp.full((L,), F32_INF, jnp.float32)
    state = (zero, inf, inf, inf, inf)
    state = lax.fori_loop(0, FULL_CHUNKS // 2, loop_body, state)

    pltpu.make_async_copy(
        vectors_hbm.at[pl.ds(base + FULL_CHUNKS * chunk_w, tail_w)],
        tailbuf.at[pl.ds(0, tail_w)], sem2).wait()
    base0 = lane * jnp.int32(DIM)
    tail_rows = [base0] * (TAIL_GROUPS - 1) + [
        jnp.minimum(lane, jnp.int32(TAIL - 1 - (TAIL_GROUPS - 1) * L))
        * jnp.int32(DIM)
    ]
    accs = _chunk_d2(tailbuf, wbuf, TAIL_GROUPS, tail_rows)
    s_sum, t0, t1, t2, t3 = state
    for g in range(TAIL_GROUPS):
        d2 = accs[g]
        valid = (lane + jnp.int32(g * L)) < jnp.int32(TAIL)
        d2m = jnp.where(valid, d2, F32_INF)
        t0, t1, t2, t3 = _insert4(t0, t1, t2, t3, d2m)
        s = _sqrt16(d2)
        s_sum = s_sum + jnp.where(valid, s, jnp.float32(0.0))

    staging[pl.ds(0, L)] = s_sum
    staging[pl.ds(L, L)] = t0
    staging[pl.ds(2 * L, L)] = t1
    staging[pl.ds(3 * L, L)] = t2
    staging[pl.ds(4 * L, L)] = t3
    pltpu.sync_copy(staging, out_hbm.at[pl.ds(wid * PER_W, PER_W)])


@functools.cache
def _sc_partials_fn():
    return pl.kernel(
        _sc_body,
        out_type=jax.ShapeDtypeStruct((NW * PER_W,), jnp.float32),
        mesh=plsc.VectorSubcoreMesh(
            core_axis_name="c", subcore_axis_name="s",
            num_cores=NC, num_subcores=NS),
        compiler_params=pltpu.CompilerParams(needs_layout_passes=False),
        scratch_types=[
            pltpu.VMEM((DIM,), jnp.float32),
            pltpu.VMEM((CHUNK * DIM,), jnp.float32),
            pltpu.VMEM((CHUNK * DIM,), jnp.float32),
            pltpu.VMEM((TAIL_GROUPS * L * DIM,), jnp.float32),
            pltpu.VMEM((PER_W,), jnp.float32),
            pltpu.SemaphoreType.DMA,
            pltpu.SemaphoreType.DMA,
            pltpu.SemaphoreType.DMA,
        ],
    )

PROWS = NW * PER_W // DIM


BR = BLK // DIM


def _tc_dist(gold_ref, v_ref, sum_o, t0_o, t1_o, t2_o, t3_o,
             sum_a, t0_a, t1_a, t2_a, t3_a):
    i = pl.program_id(0)

    @pl.when(i == 0)
    def _():
        sum_a[...] = jnp.zeros((BLK,), jnp.float32)
        for t in (t0_a, t1_a, t2_a, t3_a):
            t[...] = jnp.full((BLK,), F32_INF, jnp.float32)

    w = gold_ref[...] - jnp.float32(EPS)
    d = v_ref[...] - w
    d2 = jnp.sum(d * d, axis=1)
    rid = N_SC + i * BLK + lax.broadcasted_iota(jnp.int32, (BLK,), 0)
    valid = rid < N_ROWS
    d2m = jnp.where(valid, d2, F32_INF)

    t0, t1, t2, t3 = t0_a[...], t1_a[...], t2_a[...], t3_a[...]
    u0, u1, u2, u3 = _insert4(t0, t1, t2, t3, d2m)
    t0_a[...], t1_a[...], t2_a[...], t3_a[...] = u0, u1, u2, u3
    sum_a[...] += jnp.where(valid, jnp.sqrt(d2), jnp.float32(0.0))

    @pl.when(i == TGRID - 1)
    def _():
        sum_o[...] = sum_a[...]
        t0_o[...] = t0_a[...]
        t1_o[...] = t1_a[...]
        t2_o[...] = t2_a[...]
        t3_o[...] = t3_a[...]


def _tc_finish(p_ref, tsum_ref, tt0_ref, tt1_ref, tt2_ref, tt3_ref,
               mean_ref, topk_ref, min_ref):
    x = p_ref[...]
    r = lax.broadcasted_iota(jnp.int32, (PROWS, DIM), 0)
    c = lax.broadcasted_iota(jnp.int32, (PROWS, DIM), 1)
    f = r * DIM + c
    s = (f % PER_W) // L
    total = (jnp.sum(jnp.where(s == 0, x, jnp.float32(0.0)))
             + jnp.sum(tsum_ref[...]))
    big = jnp.int32(2**31 - 1)
    rblk = lax.broadcasted_iota(jnp.int32, (BLK,), 0)
    arrays = [[jnp.where(s >= 1, x, F32_INF), f],
              [tt0_ref[...], rblk], [tt1_ref[...], rblk],
              [tt2_ref[...], rblk], [tt3_ref[...], rblk]]
    mins = []
    for _ in range(K):
        m = arrays[0][0].min()
        for a, _i in arrays[1:]:
            m = jnp.minimum(m, a.min())
        prev = m != m
        for pair in arrays:
            a, idx = pair
            fa = jnp.min(jnp.where(a == m, idx, big))
            has = fa < big
            do = jnp.logical_and(has, jnp.logical_not(prev))
            pair[0] = jnp.where(jnp.logical_and(idx == fa, do), F32_INF, a)
            prev = jnp.logical_or(prev, has)
        mins.append(m)
    mean_ref[0, 0] = total / jnp.float32(N_ROWS)
    topk_ref[0, 0] = (jnp.sqrt(mins[0]) + jnp.sqrt(mins[1]) +
                      jnp.sqrt(mins[2]) + jnp.sqrt(mins[3])) * jnp.float32(0.25)
    min_ref[0, 0] = jnp.sqrt(mins[0])


def kernel(gold_vector, vectors):
    assert vectors.shape == (N_ROWS, DIM)
    vecs = vectors.astype(jnp.float32)
    gold = gold_vector.astype(jnp.float32)
    partials = _sc_partials_fn()(vecs.reshape(-1), gold)
    tc_outs = pl.pallas_call(
        _tc_dist,
        grid=(TGRID,),
        in_specs=[
            pl.BlockSpec((1, DIM), lambda i: (0, 0)),
            pl.BlockSpec((BLK, DIM), lambda i: (TC_BLK_OFF + i, 0)),
        ],
        out_shape=[jax.ShapeDtypeStruct((BLK,), jnp.float32)] * 5,
        out_specs=[pl.BlockSpec((BLK,), lambda i: (0,))] * 5,
        scratch_shapes=[pltpu.VMEM((BLK,), jnp.float32)] * 5,
    )(gold.reshape(1, DIM), vecs)
    mean, topk_avg, minimum = pl.pallas_call(
        _tc_finish,
        out_shape=[jax.ShapeDtypeStruct((1, 1), jnp.float32)] * 3,
        out_specs=[pl.BlockSpec(memory_space=pltpu.SMEM)] * 3,
    )(partials.reshape(PROWS, DIM), *tc_outs)
    return (mean[0, 0], topk_avg[0, 0], minimum[0, 0])
